# baseline (speedup 1.0000x reference)
import jax
import jax.numpy as jnp
from jax import lax
from jax.experimental import pallas as pl
from jax.experimental.pallas import tpu as pltpu
from jax.experimental.pallas import tpu_sc as plsc

B, S, D = 4, 8192, 1024
NC, NS = 2, 16
NW = NC * NS
SEQ_W = S // NW
R = 8
NCHUNK = SEQ_W // R
NSET = 3
LANES = 16
VPR = D // LANES


def _pos_add_body(x_hbm, pos_hbm, out_hbm, *scratch):
    p_buf = scratch[0:3]
    x_buf = scratch[3:6]
    p_sem = scratch[6:9]
    x_sem = scratch[9:12]
    wid = lax.axis_index("s") * NC + lax.axis_index("c")
    s_base = wid * SEQ_W

    def fire_loads(par, s0):
        pltpu.async_copy(pos_hbm.at[pl.ds(s0, R), :], p_buf[par], p_sem[par])
        pltpu.async_copy(
            x_hbm.at[:, pl.ds(s0, R), :], x_buf[par], x_sem[par]
        )

    def wait_loads(par, s0):
        pltpu.make_async_copy(
            pos_hbm.at[pl.ds(s0, R), :], p_buf[par], p_sem[par]
        ).wait()
        pltpu.make_async_copy(
            x_hbm.at[:, pl.ds(s0, R), :], x_buf[par], x_sem[par]
        ).wait()

    def fire_stores(par, s0):
        pltpu.async_copy(
            x_buf[par], out_hbm.at[:, pl.ds(s0, R), :], x_sem[par]
        )

    def wait_stores(par, s0):
        pltpu.make_async_copy(
            x_buf[par], out_hbm.at[:, pl.ds(s0, R), :], x_sem[par]
        ).wait()

    def add_chunk(par):
        @plsc.parallel_loop(0, R * VPR, step=1, unroll=8)
        def strip_body(k):
            i = lax.shift_right_logical(k, 6)
            j = lax.bitwise_and(k, VPR - 1)
            sl = pl.ds(j * LANES, LANES)
            pv = p_buf[par][i, sl]
            for b in range(B):
                plsc.addupdate(x_buf[par].at[b, i, sl], pv)

    def compute(par, s0):
        wait_loads(par, s0)
        add_chunk(par)
        fire_stores(par, s0)

    fire_loads(0, s_base)
    fire_loads(1, s_base + R)
    compute(0, s_base)
    fire_loads(2, s_base + 2 * R)
    compute(1, s_base + R)

    def steady_body(r, carry):
        s0 = s_base + (3 * r + 2) * R
        for k in range(NSET):
            kset = (2 + k) % NSET
            wait_stores(k, s0 + (k - 2) * R)
            fire_loads(k, s0 + (k + 1) * R)
            compute(kset, s0 + k * R)
        return carry

    lax.fori_loop(0, (NCHUNK - 5) // NSET, steady_body, 0)

    s29 = s_base + (NCHUNK - 3) * R
    wait_stores(0, s29 - 2 * R)
    fire_loads(0, s29 + R)
    compute(2, s29)
    wait_stores(1, s29 - R)
    fire_loads(1, s29 + 2 * R)
    compute(0, s29 + R)
    wait_stores(2, s29)
    compute(1, s29 + 2 * R)
    wait_stores(0, s29 + R)
    wait_stores(1, s29 + 2 * R)


def kernel(x, pos_table):
    kfn = pl.kernel(
        _pos_add_body,
        out_type=jax.ShapeDtypeStruct((B, S, D), jnp.float32),
        mesh=plsc.VectorSubcoreMesh(core_axis_name="c", subcore_axis_name="s"),
        scratch_types=(
            [pltpu.VMEM((R, D), jnp.float32) for _ in range(3)]
            + [pltpu.VMEM((B, R, D), jnp.float32) for _ in range(3)]
            + [pltpu.SemaphoreType.DMA for _ in range(6)]
        ),
    )
    return kfn(x, pos_table)


# speedup vs baseline: 1.0071x; 1.0071x over previous
import jax
import jax.numpy as jnp
from jax import lax
from jax.experimental import pallas as pl
from jax.experimental.pallas import tpu as pltpu
from jax.experimental.pallas import tpu_sc as plsc

B, S, D = 4, 8192, 1024
NC, NS = 2, 16
NW = NC * NS
SEQ_W = S // NW
R = 8
NCHUNK = SEQ_W // R
NSET = 3
LANES = 16
VPR = D // LANES


def _pos_add_body(x_hbm, pos_hbm, out_hbm, *scratch):
    p_buf = scratch[0:3]
    x_buf = scratch[3:6]
    p_sem = scratch[6:9]
    x_sem = scratch[9:12]
    wid = lax.axis_index("s") * NC + lax.axis_index("c")
    s_base = wid * SEQ_W

    def fire_loads(par, s0):
        pltpu.async_copy(pos_hbm.at[pl.ds(s0, R), :], p_buf[par], p_sem[par])
        pltpu.async_copy(
            x_hbm.at[:, pl.ds(s0, R), :], x_buf[par], x_sem[par]
        )

    def wait_loads(par, s0):
        pltpu.make_async_copy(
            pos_hbm.at[pl.ds(s0, R), :], p_buf[par], p_sem[par]
        ).wait()
        pltpu.make_async_copy(
            x_hbm.at[:, pl.ds(s0, R), :], x_buf[par], x_sem[par]
        ).wait()

    def fire_stores(par, s0):
        pltpu.async_copy(
            x_buf[par], out_hbm.at[:, pl.ds(s0, R), :], x_sem[par]
        )

    def wait_stores(par, s0):
        pltpu.make_async_copy(
            x_buf[par], out_hbm.at[:, pl.ds(s0, R), :], x_sem[par]
        ).wait()

    def add_chunk(par):
        @plsc.parallel_loop(0, R * VPR, step=1, unroll=4)
        def strip_body(k):
            i = lax.shift_right_logical(k, 6)
            j = lax.bitwise_and(k, VPR - 1)
            sl = pl.ds(j * LANES, LANES)
            pv = p_buf[par][i, sl]
            for b in range(B):
                plsc.addupdate(x_buf[par].at[b, i, sl], pv)

    def compute(par, s0):
        wait_loads(par, s0)
        add_chunk(par)
        fire_stores(par, s0)

    fire_loads(0, s_base)
    fire_loads(1, s_base + R)
    compute(0, s_base)
    fire_loads(2, s_base + 2 * R)
    compute(1, s_base + R)

    def steady_body(r, carry):
        s0 = s_base + (3 * r + 2) * R
        for k in range(NSET):
            kset = (2 + k) % NSET
            wait_stores(k, s0 + (k - 2) * R)
            fire_loads(k, s0 + (k + 1) * R)
            compute(kset, s0 + k * R)
        return carry

    lax.fori_loop(0, (NCHUNK - 5) // NSET, steady_body, 0)

    s29 = s_base + (NCHUNK - 3) * R
    wait_stores(0, s29 - 2 * R)
    fire_loads(0, s29 + R)
    compute(2, s29)
    wait_stores(1, s29 - R)
    fire_loads(1, s29 + 2 * R)
    compute(0, s29 + R)
    wait_stores(2, s29)
    compute(1, s29 + 2 * R)
    wait_stores(0, s29 + R)
    wait_stores(1, s29 + 2 * R)


def kernel(x, pos_table):
    kfn = pl.kernel(
        _pos_add_body,
        out_type=jax.ShapeDtypeStruct((B, S, D), jnp.float32),
        mesh=plsc.VectorSubcoreMesh(core_axis_name="c", subcore_axis_name="s"),
        scratch_types=(
            [pltpu.VMEM((R, D), jnp.float32) for _ in range(3)]
            + [pltpu.VMEM((B, R, D), jnp.float32) for _ in range(3)]
            + [pltpu.SemaphoreType.DMA for _ in range(6)]
        ),
    )
    return kfn(x, pos_table)
